# baseline (speedup 1.0000x reference)
import functools

import jax
import jax.numpy as jnp
from jax import lax
from jax.experimental import pallas as pl
from jax.experimental.pallas import tpu as pltpu
from jax.experimental.pallas import tpu_sc as plsc

N = 10000
E = 320000
XS = 128
GS = 16
R = 200
HEADS = 2
HID = 64
OUT = 64
ALPHA = 0.5
D = HEADS * HID

NC = 2
NS = 16
NW = NC * NS

NPAD = 10240
DEN = 2 * NPAD
RED = DEN // NS

EPT = E // NW
CH1 = 400
NCH1 = EPT // CH1
NG1 = CH1 // 16

CH2 = 64
NCH2 = E // CH2
ROWS_PT = NPAD // NS
TABR = NPAD + 256

_f32 = jnp.float32
_i32 = jnp.int32


def _leaky(v):
    return jnp.where(v >= 0, v, 0.01 * v)


@functools.partial(
    pl.kernel,
    out_type=(
        jax.ShapeDtypeStruct((E,), _f32),
        jax.ShapeDtypeStruct((E,), _f32),
        jax.ShapeDtypeStruct((NC, DEN), _f32),
    ),
    mesh=plsc.VectorSubcoreMesh(core_axis_name="c", subcore_axis_name="s",
                                num_cores=NC, num_subcores=NS),
    compiler_params=pltpu.CompilerParams(needs_layout_passes=False),
    scratch_types=(
        pltpu.VMEM((DEN,), _f32),
        pltpu.VMEM((DEN,), _f32),
        pltpu.VMEM((512,), _f32),
        pltpu.VMEM((DEN,), _f32),
        pltpu.VMEM((CH1,), _i32),
        pltpu.VMEM((CH1,), _i32),
        pltpu.VMEM((CH1,), _i32),
        pltpu.VMEM((CH1,), _f32),
        pltpu.VMEM((CH1,), _f32),
        pltpu.VMEM_SHARED((NS, DEN), _f32),
        pltpu.VMEM((RED,), _f32),
        pltpu.VMEM((RED,), _f32),
    ),
)
def _p1(src_hbm, dst_hbm, et_hbm, ssrc_hbm, sdst_hbm, sg_hbm,
        ev0_hbm, ev1_hbm, den_hbm,
        ssrc_v, sdst_v, sg_v, den_v, src_b, dst_b, et_b, ev0_b, ev1_b,
        red_sh, acc_v, tmp_v):
    c = lax.axis_index("c")
    s = lax.axis_index("s")
    wid = s * NC + c

    pltpu.sync_copy(ssrc_hbm, ssrc_v)
    pltpu.sync_copy(sdst_hbm, sdst_v)
    pltpu.sync_copy(sg_hbm, sg_v)

    zero16 = jnp.zeros((16,), _f32)

    @pl.loop(0, DEN // 16)
    def _zero(i):
        den_v[pl.ds(i * 16, 16)] = zero16

    base0 = wid * EPT

    @pl.loop(0, NCH1)
    def _chunk(ci):
        b = base0 + ci * CH1
        pltpu.sync_copy(src_hbm.at[pl.ds(b, CH1)], src_b)
        pltpu.sync_copy(dst_hbm.at[pl.ds(b, CH1)], dst_b)
        pltpu.sync_copy(et_hbm.at[pl.ds(b, CH1)], et_b)

        @pl.loop(0, NG1)
        def _grp(gi):
            o = gi * 16
            sv = src_b[pl.ds(o, 16)]
            dv = dst_b[pl.ds(o, 16)]
            tv = et_b[pl.ds(o, 16)]
            si0 = sv * 2
            di0 = dv * 2
            ti0 = tv * 2
            l0 = (plsc.load_gather(ssrc_v, [si0])
                  + plsc.load_gather(sdst_v, [di0])
                  + plsc.load_gather(sg_v, [ti0]))
            l1 = (plsc.load_gather(ssrc_v, [si0 + 1])
                  + plsc.load_gather(sdst_v, [di0 + 1])
                  + plsc.load_gather(sg_v, [ti0 + 1]))
            e0 = jnp.exp(_leaky(l0))
            e1 = jnp.exp(_leaky(l1))
            ev0_b[pl.ds(o, 16)] = e0
            ev1_b[pl.ds(o, 16)] = e1
            plsc.addupdate_scatter(den_v, [si0], e0)
            plsc.addupdate_scatter(den_v, [si0 + 1], e1)

        pltpu.sync_copy(ev0_b, ev0_hbm.at[pl.ds(b, CH1)])
        pltpu.sync_copy(ev1_b, ev1_hbm.at[pl.ds(b, CH1)])

    pltpu.sync_copy(den_v, red_sh.at[s])
    plsc.subcore_barrier()
    roff = s * RED
    pltpu.sync_copy(red_sh.at[0, pl.ds(roff, RED)], acc_v)

    @pl.loop(1, NS)
    def _red(j):
        pltpu.sync_copy(red_sh.at[j, pl.ds(roff, RED)], tmp_v)

        @pl.loop(0, RED // 16)
        def _acc(i):
            o = i * 16
            acc_v[pl.ds(o, 16)] = acc_v[pl.ds(o, 16)] + tmp_v[pl.ds(o, 16)]

    pltpu.sync_copy(acc_v, den_hbm.at[c, pl.ds(roff, RED)])


@functools.partial(
    pl.kernel,
    out_type=(
        jax.ShapeDtypeStruct((E,), _f32),
        jax.ShapeDtypeStruct((E,), _f32),
        jax.ShapeDtypeStruct((NC, DEN), _f32),
    ),
    mesh=plsc.VectorSubcoreMesh(core_axis_name="c", subcore_axis_name="s",
                                num_cores=NC, num_subcores=NS),
    compiler_params=pltpu.CompilerParams(needs_layout_passes=False),
    scratch_types=(
        pltpu.VMEM((DEN,), _f32),
        pltpu.VMEM((DEN,), _f32),
        pltpu.VMEM((CH1,), _i32),
        pltpu.VMEM((CH1,), _i32),
        pltpu.VMEM((CH1,), _f32),
        pltpu.VMEM((CH1,), _f32),
        pltpu.VMEM_SHARED((NS, DEN), _f32),
        pltpu.VMEM((RED,), _f32),
        pltpu.VMEM((RED,), _f32),
    ),
)
def _p1b(src_hbm, dst_hbm, ev0_hbm, ev1_hbm, inv_hbm,
         al0_hbm, al1_hbm, asum_hbm,
         inv_v, a_v, src_b, dst_b, e0_b, e1_b, red_sh, acc_v, tmp_v):
    c = lax.axis_index("c")
    s = lax.axis_index("s")
    wid = s * NC + c

    pltpu.sync_copy(inv_hbm, inv_v)
    zero16 = jnp.zeros((16,), _f32)

    @pl.loop(0, DEN // 16)
    def _zero(i):
        a_v[pl.ds(i * 16, 16)] = zero16

    base0 = wid * EPT

    @pl.loop(0, NCH1)
    def _chunk(ci):
        b = base0 + ci * CH1
        pltpu.sync_copy(src_hbm.at[pl.ds(b, CH1)], src_b)
        pltpu.sync_copy(dst_hbm.at[pl.ds(b, CH1)], dst_b)
        pltpu.sync_copy(ev0_hbm.at[pl.ds(b, CH1)], e0_b)
        pltpu.sync_copy(ev1_hbm.at[pl.ds(b, CH1)], e1_b)

        @pl.loop(0, NG1)
        def _grp(gi):
            o = gi * 16
            sv = src_b[pl.ds(o, 16)]
            dv = dst_b[pl.ds(o, 16)]
            si0 = sv * 2
            di0 = dv * 2
            a0 = e0_b[pl.ds(o, 16)] * plsc.load_gather(inv_v, [si0])
            a1 = e1_b[pl.ds(o, 16)] * plsc.load_gather(inv_v, [si0 + 1])
            e0_b[pl.ds(o, 16)] = a0
            e1_b[pl.ds(o, 16)] = a1
            plsc.addupdate_scatter(a_v, [di0], a0)
            plsc.addupdate_scatter(a_v, [di0 + 1], a1)

        pltpu.sync_copy(e0_b, al0_hbm.at[pl.ds(b, CH1)])
        pltpu.sync_copy(e1_b, al1_hbm.at[pl.ds(b, CH1)])

    pltpu.sync_copy(a_v, red_sh.at[s])
    plsc.subcore_barrier()
    roff = s * RED
    pltpu.sync_copy(red_sh.at[0, pl.ds(roff, RED)], acc_v)

    @pl.loop(1, NS)
    def _red(j):
        pltpu.sync_copy(red_sh.at[j, pl.ds(roff, RED)], tmp_v)

        @pl.loop(0, RED // 16)
        def _acc(i):
            o = i * 16
            acc_v[pl.ds(o, 16)] = acc_v[pl.ds(o, 16)] + tmp_v[pl.ds(o, 16)]

    pltpu.sync_copy(acc_v, asum_hbm.at[c, pl.ds(roff, RED)])


PGW = R * D


@functools.partial(
    pl.kernel,
    out_type=jax.ShapeDtypeStruct((NC, NPAD, D), _f32),
    mesh=plsc.VectorSubcoreMesh(core_axis_name="c", subcore_axis_name="s",
                                num_cores=NC, num_subcores=NS),
    compiler_params=pltpu.CompilerParams(needs_layout_passes=False),
    scratch_types=(
        pltpu.VMEM((CH2,), _i32),
        pltpu.VMEM((CH2,), _i32),
        pltpu.VMEM((CH2,), _i32),
        pltpu.VMEM((CH2,), _f32),
        pltpu.VMEM((CH2,), _f32),
        pltpu.VMEM((PGW,), _f32),
        pltpu.VMEM((CH2, D), _f32),
        pltpu.VMEM((CH2, D), _f32),
        pltpu.VMEM_SHARED((NPAD, D), _f32),
    ),
)
def _p2(src_hbm, dst_hbm, et_hbm, al0_hbm, al1_hbm, pg_hbm, tab_hbm,
        outp_hbm,
        src_b, dst_b, et_b, al0_b, al1_b, pg_v, rows_v, contrib_v,
        acc_sh):
    c = lax.axis_index("c")
    s = lax.axis_index("s")
    wid = s * NC + c
    zero16 = jnp.zeros((16,), _f32)
    iota16 = jnp.arange(16, dtype=_i32)

    pltpu.sync_copy(pg_hbm, pg_v)

    @pl.loop(0, CH2)
    def _zc(i):
        for j in range(D // 16):
            contrib_v[i, pl.ds(j * 16, 16)] = zero16

    rbase = s * ROWS_PT

    @pl.loop(0, ROWS_PT // CH2)
    def _zacc(rr):
        pltpu.sync_copy(contrib_v, acc_sh.at[pl.ds(rbase + rr * CH2, CH2), :])

    plsc.subcore_barrier()

    nchunks = jnp.where(wid < NCH2 - (NCH2 // NW) * NW, NCH2 // NW + 1,
                        NCH2 // NW)

    @pl.loop(0, nchunks)
    def _chunk(k):
        b = (wid + k * NW) * CH2
        pltpu.sync_copy(src_hbm.at[pl.ds(b, CH2)], src_b)
        pltpu.sync_copy(dst_hbm.at[pl.ds(b, CH2)], dst_b)
        pltpu.sync_copy(et_hbm.at[pl.ds(b, CH2)], et_b)
        pltpu.sync_copy(al0_hbm.at[pl.ds(b, CH2)], al0_b)
        pltpu.sync_copy(al1_hbm.at[pl.ds(b, CH2)], al1_b)

        pltpu.sync_copy(tab_hbm.at[src_b], rows_v)

        @pl.loop(0, CH2)
        def _edge(e):
            eidx = jnp.full((16,), e, _i32)
            a0v = plsc.load_gather(al0_b, [eidx])
            a1v = plsc.load_gather(al1_b, [eidx])
            etv = plsc.load_gather(et_b, [eidx]) * D
            for j in range(D // 16):
                av = a0v if j < (D // 32) else a1v
                prj = rows_v[e, pl.ds(j * 16, 16)]
                pgj = plsc.load_gather(pg_v, [etv + (j * 16 + iota16)])
                contrib_v[e, pl.ds(j * 16, 16)] = (prj + pgj) * av

        pltpu.sync_copy(contrib_v, acc_sh.at[dst_b], add=True)

    plsc.subcore_barrier()
    pltpu.sync_copy(acc_sh.at[pl.ds(rbase, ROWS_PT), :],
                    outp_hbm.at[c, pl.ds(rbase, ROWS_PT), :])


NB = 1000


def _k0_body(g_ref, wg_ref, bcat_ref, wt_ref, wrel_ref, brel_ref,
             pg4_ref, sg4_ref, gp_ref):
    g = g_ref[...]
    pg4 = jnp.dot(g, wg_ref[...], preferred_element_type=_f32,
                  precision=lax.Precision.HIGHEST) + bcat_ref[...][None, :]
    pg4_ref[...] = pg4
    sg4_ref[...] = jnp.dot(pg4, wt_ref[...], preferred_element_type=_f32,
                           precision=lax.Precision.HIGHEST)
    gp_ref[...] = jnp.dot(g, wrel_ref[...], preferred_element_type=_f32,
                          precision=lax.Precision.HIGHEST) + brel_ref[...][None, :]


def _tc_g_tables(g, wg_cat, b_cat, wt_big, wrel_t, b_rel):
    return pl.pallas_call(
        _k0_body,
        out_shape=(
            jax.ShapeDtypeStruct((R, 4 * D), _f32),
            jax.ShapeDtypeStruct((R, 8), _f32),
            jax.ShapeDtypeStruct((R, D), _f32),
        ),
    )(g, wg_cat, b_cat, wt_big, wrel_t, b_rel)


def _invden_body(di_ref, do_ref, ii_ref, io_ref):
    ii_ref[...] = 1.0 / (di_ref[0] + di_ref[1])
    io_ref[...] = 1.0 / (do_ref[0] + do_ref[1])


def _tc_invden(den_i, den_o):
    return pl.pallas_call(
        _invden_body,
        out_shape=(jax.ShapeDtypeStruct((DEN,), _f32),
                   jax.ShapeDtypeStruct((DEN,), _f32)),
    )(den_i, den_o)


def _norm_rows(v):
    nrm = jnp.sqrt(jnp.sum(v * v, axis=1, keepdims=True))
    return v / jnp.maximum(nrm, 1e-12)


def _k1_body(x_ref, w_ref, pri_ref, pci_ref, pro_ref, pco_ref, s8_ref,
             ent_ref):
    xn = _norm_rows(x_ref[...])
    big = jnp.dot(xn, w_ref[...], preferred_element_type=_f32,
                  precision=lax.Precision.HIGHEST)
    pri_ref[...] = big[:, 0:128]
    pci_ref[...] = big[:, 128:256]
    pro_ref[...] = big[:, 256:384]
    pco_ref[...] = big[:, 384:512]
    s8_ref[...] = big[:, 512:520]
    ent_ref[...] = big[:, 520:584]


def _tc_layer1(x, wcat1):
    blk = lambda w: pl.BlockSpec((NB, w), lambda i: (i, 0))
    return pl.pallas_call(
        _k1_body,
        grid=(N // NB,),
        in_specs=[blk(XS), pl.BlockSpec((XS, 584), lambda i: (0, 0))],
        out_specs=[blk(D), blk(D), blk(D), blk(D), blk(8), blk(HID)],
        out_shape=(
            jax.ShapeDtypeStruct((N, D), _f32),
            jax.ShapeDtypeStruct((N, D), _f32),
            jax.ShapeDtypeStruct((N, D), _f32),
            jax.ShapeDtypeStruct((N, D), _f32),
            jax.ShapeDtypeStruct((N, 8), _f32),
            jax.ShapeDtypeStruct((N, HID), _f32),
        ),
    )(x, wcat1)


def _combine(oi_ref, ai_ref, pci_ref, oo_ref, ao_ref, pco_ref):
    def one_dir(o_ref, a_ref, pc_ref):
        hsum = o_ref[0] + o_ref[1]
        a2 = a_ref[0] + a_ref[1]
        aexp = jnp.concatenate(
            [jnp.broadcast_to(a2[:, 0:1], (NB, HID)),
             jnp.broadcast_to(a2[:, 1:2], (NB, HID))], axis=1)
        return hsum + aexp * pc_ref[...]

    h = ALPHA * one_dir(oi_ref, ai_ref, pci_ref) + \
        (1.0 - ALPHA) * one_dir(oo_ref, ao_ref, pco_ref)
    h = _leaky(h)
    h0 = _norm_rows(h[:, 0:HID])
    h1 = _norm_rows(h[:, HID:D])
    return jnp.concatenate([h0, h1], axis=1)


def _k2_body(oi_ref, ai_ref, pci_ref, oo_ref, ao_ref, pco_ref, w_ref,
             pri_ref, pci2_ref, pro_ref, pco2_ref, s8_ref):
    h = _combine(oi_ref, ai_ref, pci_ref, oo_ref, ao_ref, pco_ref)
    big = jnp.dot(h, w_ref[...], preferred_element_type=_f32,
                  precision=lax.Precision.HIGHEST)
    pri_ref[...] = big[:, 0:128]
    pci2_ref[...] = big[:, 128:256]
    pro_ref[...] = big[:, 256:384]
    pco2_ref[...] = big[:, 384:512]
    s8_ref[...] = big[:, 512:520]


def _tc_layer2(outp_i, a_i, pc_i, outp_o, a_o, pc_o, wcat2):
    blk = lambda w: pl.BlockSpec((NB, w), lambda i: (i, 0))
    blk3 = pl.BlockSpec((NC, NB, D), lambda i: (0, i, 0))
    blka = pl.BlockSpec((NC, NB, 2), lambda i: (0, i, 0))
    return pl.pallas_call(
        _k2_body,
        grid=(N // NB,),
        in_specs=[blk3, blka, blk(D), blk3, blka, blk(D),
                  pl.BlockSpec((D, 520), lambda i: (0, 0))],
        out_specs=[blk(D), blk(D), blk(D), blk(D), blk(8)],
        out_shape=(
            jax.ShapeDtypeStruct((N, D), _f32),
            jax.ShapeDtypeStruct((N, D), _f32),
            jax.ShapeDtypeStruct((N, D), _f32),
            jax.ShapeDtypeStruct((N, D), _f32),
            jax.ShapeDtypeStruct((N, 8), _f32),
        ),
    )(outp_i, a_i, pc_i, outp_o, a_o, pc_o, wcat2)


def _k3_body(oi_ref, ai_ref, pci_ref, oo_ref, ao_ref, pco_ref, ent_ref,
             bent_ref, out_ref):
    h = _combine(oi_ref, ai_ref, pci_ref, oo_ref, ao_ref, pco_ref)
    ent = ent_ref[...] + bent_ref[...][None, :]
    hp = h + jnp.concatenate([ent, ent], axis=1)
    out_ref[...] = _norm_rows(hp)


def _tc_final(outp_i, a_i, pc_i, outp_o, a_o, pc_o, ent, b_ent):
    blk = lambda w: pl.BlockSpec((NB, w), lambda i: (i, 0))
    blk3 = pl.BlockSpec((NC, NB, D), lambda i: (0, i, 0))
    blka = pl.BlockSpec((NC, NB, 2), lambda i: (0, i, 0))
    return pl.pallas_call(
        _k3_body,
        grid=(N // NB,),
        in_specs=[blk3, blka, blk(D), blk3, blka, blk(D), blk(HID),
                  pl.BlockSpec((HID,), lambda i: (0,))],
        out_specs=blk(D),
        out_shape=jax.ShapeDtypeStruct((N, D), _f32),
    )(outp_i, a_i, pc_i, outp_o, a_o, pc_o, ent, b_ent)


def _wt_block(att):
    z = jnp.zeros((HID, 1), _f32)
    c0 = jnp.concatenate([att[0, 0][:, None], z], axis=0)
    c1 = jnp.concatenate([z, att[0, 1][:, None]], axis=0)
    return jnp.concatenate([c0, c1], axis=1)


def _split_w(W):
    dh = (W.shape[1] - GS) // 2
    return W[:, 0:dh].T, W[:, dh:2 * dh].T, W[:, 2 * dh:].T


def _layer_tables(wt_in, wt_out, W_in, W_out):
    wr_i, wc_i, _ = _split_w(W_in)
    wr_o, wc_o, _ = _split_w(W_out)
    cols = [wr_i, wc_i, wr_o, wc_o,
            wr_i @ wt_in, wc_i @ wt_in, wr_o @ wt_out, wc_o @ wt_out]
    return jnp.concatenate(cols, axis=1)


def _mk_tab(pr, pg):
    top = jnp.concatenate([pr, jnp.zeros((NPAD - N, D), _f32)], axis=0)
    bot = jnp.concatenate(
        [pg, jnp.zeros((TABR - NPAD - R, D), _f32)], axis=0)
    return jnp.concatenate([top, bot], axis=0)


def _run_sc_layer(row, col, et, s8, sg4, sg_cols, pr_i, pc_i, pr_o, pc_o,
                  pg_i, pg_o):
    def _padto(v, L):
        return jnp.concatenate([v, jnp.zeros((L - v.shape[0],), _f32)])

    sr_i = _padto(s8[:, 0:2].reshape(-1), DEN)
    sc_i = _padto(s8[:, 2:4].reshape(-1), DEN)
    sr_o = _padto(s8[:, 4:6].reshape(-1), DEN)
    sc_o = _padto(s8[:, 6:8].reshape(-1), DEN)
    sg_i = _padto(sg4[:, sg_cols[0]:sg_cols[0] + 2].reshape(-1), 512)
    sg_o = _padto(sg4[:, sg_cols[1]:sg_cols[1] + 2].reshape(-1), 512)

    ev0_i, ev1_i, den_i = _p1(row, col, et, sr_i, sc_i, sg_i)
    ev0_o, ev1_o, den_o = _p1(col, row, et, sr_o, sc_o, sg_o)
    inv_i, inv_o = _tc_invden(den_i, den_o)
    al0_i, al1_i, a_i = _p1b(row, col, ev0_i, ev1_i, inv_i)
    al0_o, al1_o, a_o = _p1b(col, row, ev0_o, ev1_o, inv_o)
    outp_i = _p2(row, col, et, al0_i, al1_i, pg_i.reshape(-1),
                 _mk_tab(pr_i, pg_i))
    outp_o = _p2(col, row, et, al0_o, al1_o, pg_o.reshape(-1),
                 _mk_tab(pr_o, pg_o))
    return (outp_i, a_i.reshape(NC, NPAD, 2), pc_i,
            outp_o, a_o.reshape(NC, NPAD, 2), pc_o)


def kernel(x, g, edge_idx, edge_type, W_in1, b_in1, att_in1, W_out1, b_out1,
           att_out1, W_in2, b_in2, att_in2, W_out2, b_out2, att_out2,
           W_ent, b_ent, W_rel, b_rel):
    row = edge_idx[0]
    col = edge_idx[1]
    et = edge_type

    wt_in1 = _wt_block(att_in1)
    wt_out1 = _wt_block(att_out1)
    wt_in2 = _wt_block(att_in2)
    wt_out2 = _wt_block(att_out2)

    wcat1 = jnp.concatenate(
        [_layer_tables(wt_in1, wt_out1, W_in1, W_out1), W_ent.T], axis=1)
    wcat2 = _layer_tables(wt_in2, wt_out2, W_in2, W_out2)

    _, _, wg_i1 = _split_w(W_in1)
    _, _, wg_o1 = _split_w(W_out1)
    _, _, wg_i2 = _split_w(W_in2)
    _, _, wg_o2 = _split_w(W_out2)
    wg_cat = jnp.concatenate([wg_i1, wg_o1, wg_i2, wg_o2], axis=1)
    b_cat = jnp.concatenate([b_in1, b_out1, b_in2, b_out2])
    z2 = jnp.zeros((D, 2), _f32)

    def blkdiag(w0, w1, w2, w3):
        def rowblk(i, w):
            pre = [z2] * i
            post = [z2] * (3 - i)
            return jnp.concatenate(pre + [w] + post, axis=1)
        return jnp.concatenate(
            [rowblk(0, w0), rowblk(1, w1), rowblk(2, w2), rowblk(3, w3)],
            axis=0)

    wt_big = blkdiag(wt_in1, wt_out1, wt_in2, wt_out2)

    pg4, sg4, g_prime = _tc_g_tables(g, wg_cat, b_cat, wt_big, W_rel.T, b_rel)

    pr_i1, pc_i1, pr_o1, pc_o1, s8_1, ent = _tc_layer1(x, wcat1)
    sc_args1 = _run_sc_layer(
        row, col, et, s8_1, sg4, (0, 2), pr_i1, pc_i1, pr_o1, pc_o1,
        pg4[:, 0:128], pg4[:, 128:256])

    pr_i2, pc_i2, pr_o2, pc_o2, s8_2 = _tc_layer2(*sc_args1, wcat2)
    sc_args2 = _run_sc_layer(
        row, col, et, s8_2, sg4, (4, 6), pr_i2, pc_i2, pr_o2, pc_o2,
        pg4[:, 256:384], pg4[:, 384:512])

    h_prime = _tc_final(*sc_args2, ent, b_ent)
    return (h_prime, g_prime)


# speedup vs baseline: 1.0187x; 1.0187x over previous
import functools

import jax
import jax.numpy as jnp
from jax import lax
from jax.experimental import pallas as pl
from jax.experimental.pallas import tpu as pltpu
from jax.experimental.pallas import tpu_sc as plsc

N = 10000
E = 320000
XS = 128
GS = 16
R = 200
HEADS = 2
HID = 64
OUT = 64
ALPHA = 0.5
D = HEADS * HID

NC = 2
NS = 16
NW = NC * NS

NPAD = 10240
DEN = 2 * NPAD
RED = DEN // NS

EPT = E // NW
CH1 = 400
NCH1 = EPT // CH1
NG1 = CH1 // 16

CH2 = 64
NCH2 = E // CH2
ROWS_PT = NPAD // NS
TABR = NPAD + 256

_f32 = jnp.float32
_i32 = jnp.int32


def _leaky(v):
    return jnp.where(v >= 0, v, 0.01 * v)


@functools.partial(
    pl.kernel,
    out_type=(
        jax.ShapeDtypeStruct((E,), _f32),
        jax.ShapeDtypeStruct((E,), _f32),
        jax.ShapeDtypeStruct((NC, DEN), _f32),
    ),
    mesh=plsc.VectorSubcoreMesh(core_axis_name="c", subcore_axis_name="s",
                                num_cores=NC, num_subcores=NS),
    compiler_params=pltpu.CompilerParams(needs_layout_passes=False),
    scratch_types=(
        pltpu.VMEM((DEN,), _f32),
        pltpu.VMEM((DEN,), _f32),
        pltpu.VMEM((512,), _f32),
        pltpu.VMEM((DEN,), _f32),
        pltpu.VMEM((CH1,), _i32),
        pltpu.VMEM((CH1,), _i32),
        pltpu.VMEM((CH1,), _i32),
        pltpu.VMEM((CH1,), _f32),
        pltpu.VMEM((CH1,), _f32),
        pltpu.VMEM_SHARED((NS, DEN), _f32),
        pltpu.VMEM((RED,), _f32),
        pltpu.VMEM((RED,), _f32),
    ),
)
def _p1(src_hbm, dst_hbm, et_hbm, ssrc_hbm, sdst_hbm, sg_hbm,
        ev0_hbm, ev1_hbm, den_hbm,
        ssrc_v, sdst_v, sg_v, den_v, src_b, dst_b, et_b, ev0_b, ev1_b,
        red_sh, acc_v, tmp_v):
    c = lax.axis_index("c")
    s = lax.axis_index("s")
    wid = s * NC + c

    pltpu.sync_copy(ssrc_hbm, ssrc_v)
    pltpu.sync_copy(sdst_hbm, sdst_v)
    pltpu.sync_copy(sg_hbm, sg_v)

    zero16 = jnp.zeros((16,), _f32)

    @pl.loop(0, DEN // 16)
    def _zero(i):
        den_v[pl.ds(i * 16, 16)] = zero16

    base0 = wid * EPT

    @pl.loop(0, NCH1)
    def _chunk(ci):
        b = base0 + ci * CH1
        pltpu.sync_copy(src_hbm.at[pl.ds(b, CH1)], src_b)
        pltpu.sync_copy(dst_hbm.at[pl.ds(b, CH1)], dst_b)
        pltpu.sync_copy(et_hbm.at[pl.ds(b, CH1)], et_b)

        @pl.loop(0, NG1)
        def _grp(gi):
            o = gi * 16
            sv = src_b[pl.ds(o, 16)]
            dv = dst_b[pl.ds(o, 16)]
            tv = et_b[pl.ds(o, 16)]
            si0 = sv * 2
            di0 = dv * 2
            ti0 = tv * 2
            l0 = (plsc.load_gather(ssrc_v, [si0])
                  + plsc.load_gather(sdst_v, [di0])
                  + plsc.load_gather(sg_v, [ti0]))
            l1 = (plsc.load_gather(ssrc_v, [si0 + 1])
                  + plsc.load_gather(sdst_v, [di0 + 1])
                  + plsc.load_gather(sg_v, [ti0 + 1]))
            e0 = jnp.exp(_leaky(l0))
            e1 = jnp.exp(_leaky(l1))
            ev0_b[pl.ds(o, 16)] = e0
            ev1_b[pl.ds(o, 16)] = e1
            plsc.addupdate_scatter(den_v, [si0], e0)
            plsc.addupdate_scatter(den_v, [si0 + 1], e1)

        pltpu.sync_copy(ev0_b, ev0_hbm.at[pl.ds(b, CH1)])
        pltpu.sync_copy(ev1_b, ev1_hbm.at[pl.ds(b, CH1)])

    pltpu.sync_copy(den_v, red_sh.at[s])
    plsc.subcore_barrier()
    roff = s * RED
    pltpu.sync_copy(red_sh.at[0, pl.ds(roff, RED)], acc_v)

    @pl.loop(1, NS)
    def _red(j):
        pltpu.sync_copy(red_sh.at[j, pl.ds(roff, RED)], tmp_v)

        @pl.loop(0, RED // 16)
        def _acc(i):
            o = i * 16
            acc_v[pl.ds(o, 16)] = acc_v[pl.ds(o, 16)] + tmp_v[pl.ds(o, 16)]

    pltpu.sync_copy(acc_v, den_hbm.at[c, pl.ds(roff, RED)])


@functools.partial(
    pl.kernel,
    out_type=(
        jax.ShapeDtypeStruct((E,), _f32),
        jax.ShapeDtypeStruct((E,), _f32),
        jax.ShapeDtypeStruct((NC, DEN), _f32),
    ),
    mesh=plsc.VectorSubcoreMesh(core_axis_name="c", subcore_axis_name="s",
                                num_cores=NC, num_subcores=NS),
    compiler_params=pltpu.CompilerParams(needs_layout_passes=False),
    scratch_types=(
        pltpu.VMEM((DEN,), _f32),
        pltpu.VMEM((DEN,), _f32),
        pltpu.VMEM((CH1,), _i32),
        pltpu.VMEM((CH1,), _i32),
        pltpu.VMEM((CH1,), _f32),
        pltpu.VMEM((CH1,), _f32),
        pltpu.VMEM_SHARED((NS, DEN), _f32),
        pltpu.VMEM((RED,), _f32),
        pltpu.VMEM((RED,), _f32),
    ),
)
def _p1b(src_hbm, dst_hbm, ev0_hbm, ev1_hbm, inv_hbm,
         al0_hbm, al1_hbm, asum_hbm,
         inv_v, a_v, src_b, dst_b, e0_b, e1_b, red_sh, acc_v, tmp_v):
    c = lax.axis_index("c")
    s = lax.axis_index("s")
    wid = s * NC + c

    pltpu.sync_copy(inv_hbm, inv_v)
    zero16 = jnp.zeros((16,), _f32)

    @pl.loop(0, DEN // 16)
    def _zero(i):
        a_v[pl.ds(i * 16, 16)] = zero16

    base0 = wid * EPT

    @pl.loop(0, NCH1)
    def _chunk(ci):
        b = base0 + ci * CH1
        pltpu.sync_copy(src_hbm.at[pl.ds(b, CH1)], src_b)
        pltpu.sync_copy(dst_hbm.at[pl.ds(b, CH1)], dst_b)
        pltpu.sync_copy(ev0_hbm.at[pl.ds(b, CH1)], e0_b)
        pltpu.sync_copy(ev1_hbm.at[pl.ds(b, CH1)], e1_b)

        @pl.loop(0, NG1)
        def _grp(gi):
            o = gi * 16
            sv = src_b[pl.ds(o, 16)]
            dv = dst_b[pl.ds(o, 16)]
            si0 = sv * 2
            di0 = dv * 2
            a0 = e0_b[pl.ds(o, 16)] * plsc.load_gather(inv_v, [si0])
            a1 = e1_b[pl.ds(o, 16)] * plsc.load_gather(inv_v, [si0 + 1])
            e0_b[pl.ds(o, 16)] = a0
            e1_b[pl.ds(o, 16)] = a1
            plsc.addupdate_scatter(a_v, [di0], a0)
            plsc.addupdate_scatter(a_v, [di0 + 1], a1)

        pltpu.sync_copy(e0_b, al0_hbm.at[pl.ds(b, CH1)])
        pltpu.sync_copy(e1_b, al1_hbm.at[pl.ds(b, CH1)])

    pltpu.sync_copy(a_v, red_sh.at[s])
    plsc.subcore_barrier()
    roff = s * RED
    pltpu.sync_copy(red_sh.at[0, pl.ds(roff, RED)], acc_v)

    @pl.loop(1, NS)
    def _red(j):
        pltpu.sync_copy(red_sh.at[j, pl.ds(roff, RED)], tmp_v)

        @pl.loop(0, RED // 16)
        def _acc(i):
            o = i * 16
            acc_v[pl.ds(o, 16)] = acc_v[pl.ds(o, 16)] + tmp_v[pl.ds(o, 16)]

    pltpu.sync_copy(acc_v, asum_hbm.at[c, pl.ds(roff, RED)])


@functools.partial(
    pl.kernel,
    out_type=jax.ShapeDtypeStruct((NC, NPAD, D), _f32),
    mesh=plsc.VectorSubcoreMesh(core_axis_name="c", subcore_axis_name="s",
                                num_cores=NC, num_subcores=NS),
    compiler_params=pltpu.CompilerParams(needs_layout_passes=False),
    scratch_types=(
        pltpu.VMEM((CH2,), _i32),
        pltpu.VMEM((CH2,), _i32),
        pltpu.VMEM((CH2,), _i32),
        pltpu.VMEM((CH2,), _f32),
        pltpu.VMEM((CH2,), _f32),
        pltpu.VMEM((2 * CH2,), _i32),
        pltpu.VMEM((2 * CH2, D), _f32),
        pltpu.VMEM((CH2, D), _f32),
        pltpu.VMEM_SHARED((NPAD, D), _f32),
    ),
)
def _p2(src_hbm, dst_hbm, et_hbm, al0_hbm, al1_hbm, tab_hbm,
        outp_hbm,
        src_b, dst_b, et_b, al0_b, al1_b, idx_b, rows_v, contrib_v,
        acc_sh):
    c = lax.axis_index("c")
    s = lax.axis_index("s")
    wid = s * NC + c
    zero16 = jnp.zeros((16,), _f32)
    npad16 = jnp.full((16,), NPAD, _i32)

    @pl.loop(0, CH2)
    def _zc(i):
        for j in range(D // 16):
            contrib_v[i, pl.ds(j * 16, 16)] = zero16

    rbase = s * ROWS_PT

    @pl.loop(0, ROWS_PT // CH2)
    def _zacc(rr):
        pltpu.sync_copy(contrib_v, acc_sh.at[pl.ds(rbase + rr * CH2, CH2), :])

    plsc.subcore_barrier()

    nchunks = jnp.where(wid < NCH2 - (NCH2 // NW) * NW, NCH2 // NW + 1,
                        NCH2 // NW)

    @pl.loop(0, nchunks)
    def _chunk(k):
        b = (wid + k * NW) * CH2
        pltpu.sync_copy(src_hbm.at[pl.ds(b, CH2)], src_b)
        pltpu.sync_copy(dst_hbm.at[pl.ds(b, CH2)], dst_b)
        pltpu.sync_copy(et_hbm.at[pl.ds(b, CH2)], et_b)
        pltpu.sync_copy(al0_hbm.at[pl.ds(b, CH2)], al0_b)
        pltpu.sync_copy(al1_hbm.at[pl.ds(b, CH2)], al1_b)

        @pl.loop(0, CH2 // 16)
        def _bi(gi):
            o = gi * 16
            idx_b[pl.ds(o, 16)] = src_b[pl.ds(o, 16)]
            idx_b[pl.ds(CH2 + o, 16)] = et_b[pl.ds(o, 16)] + npad16

        pltpu.sync_copy(tab_hbm.at[idx_b], rows_v)

        @pl.loop(0, CH2)
        def _edge(e):
            eidx = jnp.full((16,), e, _i32)
            a0v = plsc.load_gather(al0_b, [eidx])
            a1v = plsc.load_gather(al1_b, [eidx])
            for j in range(D // 16):
                av = a0v if j < (D // 32) else a1v
                prj = rows_v[e, pl.ds(j * 16, 16)]
                pgj = rows_v[CH2 + e, pl.ds(j * 16, 16)]
                contrib_v[e, pl.ds(j * 16, 16)] = (prj + pgj) * av

        pltpu.sync_copy(contrib_v, acc_sh.at[dst_b], add=True)

    plsc.subcore_barrier()
    pltpu.sync_copy(acc_sh.at[pl.ds(rbase, ROWS_PT), :],
                    outp_hbm.at[c, pl.ds(rbase, ROWS_PT), :])


NB = 1000


def _k0_body(g_ref, wg_ref, bcat_ref, wt_ref, wrel_ref, brel_ref,
             pg4_ref, sg4_ref, gp_ref):
    g = g_ref[...]
    pg4 = jnp.dot(g, wg_ref[...], preferred_element_type=_f32,
                  precision=lax.Precision.HIGHEST) + bcat_ref[...][None, :]
    pg4_ref[...] = pg4
    sg4_ref[...] = jnp.dot(pg4, wt_ref[...], preferred_element_type=_f32,
                           precision=lax.Precision.HIGHEST)
    gp_ref[...] = jnp.dot(g, wrel_ref[...], preferred_element_type=_f32,
                          precision=lax.Precision.HIGHEST) + brel_ref[...][None, :]


def _tc_g_tables(g, wg_cat, b_cat, wt_big, wrel_t, b_rel):
    return pl.pallas_call(
        _k0_body,
        out_shape=(
            jax.ShapeDtypeStruct((R, 4 * D), _f32),
            jax.ShapeDtypeStruct((R, 8), _f32),
            jax.ShapeDtypeStruct((R, D), _f32),
        ),
    )(g, wg_cat, b_cat, wt_big, wrel_t, b_rel)


def _invden_body(di_ref, do_ref, ii_ref, io_ref):
    ii_ref[...] = 1.0 / (di_ref[0] + di_ref[1])
    io_ref[...] = 1.0 / (do_ref[0] + do_ref[1])


def _tc_invden(den_i, den_o):
    return pl.pallas_call(
        _invden_body,
        out_shape=(jax.ShapeDtypeStruct((DEN,), _f32),
                   jax.ShapeDtypeStruct((DEN,), _f32)),
    )(den_i, den_o)


def _norm_rows(v):
    nrm = jnp.sqrt(jnp.sum(v * v, axis=1, keepdims=True))
    return v / jnp.maximum(nrm, 1e-12)


def _k1_body(x_ref, w_ref, pri_ref, pci_ref, pro_ref, pco_ref, s8_ref,
             ent_ref):
    xn = _norm_rows(x_ref[...])
    big = jnp.dot(xn, w_ref[...], preferred_element_type=_f32,
                  precision=lax.Precision.HIGHEST)
    pri_ref[...] = big[:, 0:128]
    pci_ref[...] = big[:, 128:256]
    pro_ref[...] = big[:, 256:384]
    pco_ref[...] = big[:, 384:512]
    s8_ref[...] = big[:, 512:520]
    ent_ref[...] = big[:, 520:584]


def _tc_layer1(x, wcat1):
    blk = lambda w: pl.BlockSpec((NB, w), lambda i: (i, 0))
    return pl.pallas_call(
        _k1_body,
        grid=(N // NB,),
        in_specs=[blk(XS), pl.BlockSpec((XS, 584), lambda i: (0, 0))],
        out_specs=[blk(D), blk(D), blk(D), blk(D), blk(8), blk(HID)],
        out_shape=(
            jax.ShapeDtypeStruct((N, D), _f32),
            jax.ShapeDtypeStruct((N, D), _f32),
            jax.ShapeDtypeStruct((N, D), _f32),
            jax.ShapeDtypeStruct((N, D), _f32),
            jax.ShapeDtypeStruct((N, 8), _f32),
            jax.ShapeDtypeStruct((N, HID), _f32),
        ),
    )(x, wcat1)


def _combine(oi_ref, ai_ref, pci_ref, oo_ref, ao_ref, pco_ref):
    def one_dir(o_ref, a_ref, pc_ref):
        hsum = o_ref[0] + o_ref[1]
        a2 = a_ref[0] + a_ref[1]
        aexp = jnp.concatenate(
            [jnp.broadcast_to(a2[:, 0:1], (NB, HID)),
             jnp.broadcast_to(a2[:, 1:2], (NB, HID))], axis=1)
        return hsum + aexp * pc_ref[...]

    h = ALPHA * one_dir(oi_ref, ai_ref, pci_ref) + \
        (1.0 - ALPHA) * one_dir(oo_ref, ao_ref, pco_ref)
    h = _leaky(h)
    h0 = _norm_rows(h[:, 0:HID])
    h1 = _norm_rows(h[:, HID:D])
    return jnp.concatenate([h0, h1], axis=1)


def _k2_body(oi_ref, ai_ref, pci_ref, oo_ref, ao_ref, pco_ref, w_ref,
             pri_ref, pci2_ref, pro_ref, pco2_ref, s8_ref):
    h = _combine(oi_ref, ai_ref, pci_ref, oo_ref, ao_ref, pco_ref)
    big = jnp.dot(h, w_ref[...], preferred_element_type=_f32,
                  precision=lax.Precision.HIGHEST)
    pri_ref[...] = big[:, 0:128]
    pci2_ref[...] = big[:, 128:256]
    pro_ref[...] = big[:, 256:384]
    pco2_ref[...] = big[:, 384:512]
    s8_ref[...] = big[:, 512:520]


def _tc_layer2(outp_i, a_i, pc_i, outp_o, a_o, pc_o, wcat2):
    blk = lambda w: pl.BlockSpec((NB, w), lambda i: (i, 0))
    blk3 = pl.BlockSpec((NC, NB, D), lambda i: (0, i, 0))
    blka = pl.BlockSpec((NC, NB, 2), lambda i: (0, i, 0))
    return pl.pallas_call(
        _k2_body,
        grid=(N // NB,),
        in_specs=[blk3, blka, blk(D), blk3, blka, blk(D),
                  pl.BlockSpec((D, 520), lambda i: (0, 0))],
        out_specs=[blk(D), blk(D), blk(D), blk(D), blk(8)],
        out_shape=(
            jax.ShapeDtypeStruct((N, D), _f32),
            jax.ShapeDtypeStruct((N, D), _f32),
            jax.ShapeDtypeStruct((N, D), _f32),
            jax.ShapeDtypeStruct((N, D), _f32),
            jax.ShapeDtypeStruct((N, 8), _f32),
        ),
    )(outp_i, a_i, pc_i, outp_o, a_o, pc_o, wcat2)


def _k3_body(oi_ref, ai_ref, pci_ref, oo_ref, ao_ref, pco_ref, ent_ref,
             bent_ref, out_ref):
    h = _combine(oi_ref, ai_ref, pci_ref, oo_ref, ao_ref, pco_ref)
    ent = ent_ref[...] + bent_ref[...][None, :]
    hp = h + jnp.concatenate([ent, ent], axis=1)
    out_ref[...] = _norm_rows(hp)


def _tc_final(outp_i, a_i, pc_i, outp_o, a_o, pc_o, ent, b_ent):
    blk = lambda w: pl.BlockSpec((NB, w), lambda i: (i, 0))
    blk3 = pl.BlockSpec((NC, NB, D), lambda i: (0, i, 0))
    blka = pl.BlockSpec((NC, NB, 2), lambda i: (0, i, 0))
    return pl.pallas_call(
        _k3_body,
        grid=(N // NB,),
        in_specs=[blk3, blka, blk(D), blk3, blka, blk(D), blk(HID),
                  pl.BlockSpec((HID,), lambda i: (0,))],
        out_specs=blk(D),
        out_shape=jax.ShapeDtypeStruct((N, D), _f32),
    )(outp_i, a_i, pc_i, outp_o, a_o, pc_o, ent, b_ent)


def _wt_block(att):
    z = jnp.zeros((HID, 1), _f32)
    c0 = jnp.concatenate([att[0, 0][:, None], z], axis=0)
    c1 = jnp.concatenate([z, att[0, 1][:, None]], axis=0)
    return jnp.concatenate([c0, c1], axis=1)


def _split_w(W):
    dh = (W.shape[1] - GS) // 2
    return W[:, 0:dh].T, W[:, dh:2 * dh].T, W[:, 2 * dh:].T


def _layer_tables(wt_in, wt_out, W_in, W_out):
    wr_i, wc_i, _ = _split_w(W_in)
    wr_o, wc_o, _ = _split_w(W_out)
    cols = [wr_i, wc_i, wr_o, wc_o,
            wr_i @ wt_in, wc_i @ wt_in, wr_o @ wt_out, wc_o @ wt_out]
    return jnp.concatenate(cols, axis=1)


def _mk_tab(pr, pg):
    top = jnp.concatenate([pr, jnp.zeros((NPAD - N, D), _f32)], axis=0)
    bot = jnp.concatenate(
        [pg, jnp.zeros((TABR - NPAD - R, D), _f32)], axis=0)
    return jnp.concatenate([top, bot], axis=0)


def _run_sc_layer(row, col, et, s8, sg4, sg_cols, pr_i, pc_i, pr_o, pc_o,
                  pg_i, pg_o):
    def _padto(v, L):
        return jnp.concatenate([v, jnp.zeros((L - v.shape[0],), _f32)])

    sr_i = _padto(s8[:, 0:2].reshape(-1), DEN)
    sc_i = _padto(s8[:, 2:4].reshape(-1), DEN)
    sr_o = _padto(s8[:, 4:6].reshape(-1), DEN)
    sc_o = _padto(s8[:, 6:8].reshape(-1), DEN)
    sg_i = _padto(sg4[:, sg_cols[0]:sg_cols[0] + 2].reshape(-1), 512)
    sg_o = _padto(sg4[:, sg_cols[1]:sg_cols[1] + 2].reshape(-1), 512)

    ev0_i, ev1_i, den_i = _p1(row, col, et, sr_i, sc_i, sg_i)
    ev0_o, ev1_o, den_o = _p1(col, row, et, sr_o, sc_o, sg_o)
    inv_i, inv_o = _tc_invden(den_i, den_o)
    al0_i, al1_i, a_i = _p1b(row, col, ev0_i, ev1_i, inv_i)
    al0_o, al1_o, a_o = _p1b(col, row, ev0_o, ev1_o, inv_o)
    outp_i = _p2(row, col, et, al0_i, al1_i, _mk_tab(pr_i, pg_i))
    outp_o = _p2(col, row, et, al0_o, al1_o, _mk_tab(pr_o, pg_o))
    return (outp_i, a_i.reshape(NC, NPAD, 2), pc_i,
            outp_o, a_o.reshape(NC, NPAD, 2), pc_o)


def kernel(x, g, edge_idx, edge_type, W_in1, b_in1, att_in1, W_out1, b_out1,
           att_out1, W_in2, b_in2, att_in2, W_out2, b_out2, att_out2,
           W_ent, b_ent, W_rel, b_rel):
    row = edge_idx[0]
    col = edge_idx[1]
    et = edge_type

    wt_in1 = _wt_block(att_in1)
    wt_out1 = _wt_block(att_out1)
    wt_in2 = _wt_block(att_in2)
    wt_out2 = _wt_block(att_out2)

    wcat1 = jnp.concatenate(
        [_layer_tables(wt_in1, wt_out1, W_in1, W_out1), W_ent.T], axis=1)
    wcat2 = _layer_tables(wt_in2, wt_out2, W_in2, W_out2)

    _, _, wg_i1 = _split_w(W_in1)
    _, _, wg_o1 = _split_w(W_out1)
    _, _, wg_i2 = _split_w(W_in2)
    _, _, wg_o2 = _split_w(W_out2)
    wg_cat = jnp.concatenate([wg_i1, wg_o1, wg_i2, wg_o2], axis=1)
    b_cat = jnp.concatenate([b_in1, b_out1, b_in2, b_out2])
    z2 = jnp.zeros((D, 2), _f32)

    def blkdiag(w0, w1, w2, w3):
        def rowblk(i, w):
            pre = [z2] * i
            post = [z2] * (3 - i)
            return jnp.concatenate(pre + [w] + post, axis=1)
        return jnp.concatenate(
            [rowblk(0, w0), rowblk(1, w1), rowblk(2, w2), rowblk(3, w3)],
            axis=0)

    wt_big = blkdiag(wt_in1, wt_out1, wt_in2, wt_out2)

    pg4, sg4, g_prime = _tc_g_tables(g, wg_cat, b_cat, wt_big, W_rel.T, b_rel)

    pr_i1, pc_i1, pr_o1, pc_o1, s8_1, ent = _tc_layer1(x, wcat1)
    sc_args1 = _run_sc_layer(
        row, col, et, s8_1, sg4, (0, 2), pr_i1, pc_i1, pr_o1, pc_o1,
        pg4[:, 0:128], pg4[:, 128:256])

    pr_i2, pc_i2, pr_o2, pc_o2, s8_2 = _tc_layer2(*sc_args1, wcat2)
    sc_args2 = _run_sc_layer(
        row, col, et, s8_2, sg4, (4, 6), pr_i2, pc_i2, pr_o2, pc_o2,
        pg4[:, 256:384], pg4[:, 384:512])

    h_prime = _tc_final(*sc_args2, ent, b_ent)
    return (h_prime, g_prime)


# speedup vs baseline: 1.0917x; 1.0716x over previous
import functools

import jax
import jax.numpy as jnp
from jax import lax
from jax.experimental import pallas as pl
from jax.experimental.pallas import tpu as pltpu
from jax.experimental.pallas import tpu_sc as plsc

N = 10000
E = 320000
XS = 128
GS = 16
R = 200
HEADS = 2
HID = 64
OUT = 64
ALPHA = 0.5
D = HEADS * HID

NC = 2
NS = 16
NW = NC * NS

NPAD = 10240
DEN = 2 * NPAD
RED = DEN // NS

EPT = E // NW
CH1 = 400
NCH1 = EPT // CH1
NG1 = CH1 // 16

CH2 = 80
NCH2 = E // CH2
ROWS_PT = NPAD // NS
TABR = NPAD + 256

_f32 = jnp.float32
_i32 = jnp.int32


def _leaky(v):
    return jnp.where(v >= 0, v, 0.01 * v)


@functools.partial(
    pl.kernel,
    out_type=(
        jax.ShapeDtypeStruct((E,), _f32),
        jax.ShapeDtypeStruct((E,), _f32),
        jax.ShapeDtypeStruct((NC, DEN), _f32),
    ),
    mesh=plsc.VectorSubcoreMesh(core_axis_name="c", subcore_axis_name="s",
                                num_cores=NC, num_subcores=NS),
    compiler_params=pltpu.CompilerParams(needs_layout_passes=False),
    scratch_types=(
        pltpu.VMEM((DEN,), _f32),
        pltpu.VMEM((DEN,), _f32),
        pltpu.VMEM((512,), _f32),
        pltpu.VMEM((DEN,), _f32),
        pltpu.VMEM((CH1,), _i32),
        pltpu.VMEM((CH1,), _i32),
        pltpu.VMEM((CH1,), _i32),
        pltpu.VMEM((CH1,), _f32),
        pltpu.VMEM((CH1,), _f32),
        pltpu.VMEM_SHARED((NS, DEN), _f32),
        pltpu.VMEM((RED,), _f32),
        pltpu.VMEM((RED,), _f32),
    ),
)
def _p1(src_hbm, dst_hbm, et_hbm, ssrc_hbm, sdst_hbm, sg_hbm,
        ev0_hbm, ev1_hbm, den_hbm,
        ssrc_v, sdst_v, sg_v, den_v, src_b, dst_b, et_b, ev0_b, ev1_b,
        red_sh, acc_v, tmp_v):
    c = lax.axis_index("c")
    s = lax.axis_index("s")
    wid = s * NC + c

    pltpu.sync_copy(ssrc_hbm, ssrc_v)
    pltpu.sync_copy(sdst_hbm, sdst_v)
    pltpu.sync_copy(sg_hbm, sg_v)

    zero16 = jnp.zeros((16,), _f32)

    @pl.loop(0, DEN // 16)
    def _zero(i):
        den_v[pl.ds(i * 16, 16)] = zero16

    base0 = wid * EPT

    @pl.loop(0, NCH1)
    def _chunk(ci):
        b = base0 + ci * CH1
        pltpu.sync_copy(src_hbm.at[pl.ds(b, CH1)], src_b)
        pltpu.sync_copy(dst_hbm.at[pl.ds(b, CH1)], dst_b)
        pltpu.sync_copy(et_hbm.at[pl.ds(b, CH1)], et_b)

        @pl.loop(0, NG1)
        def _grp(gi):
            o = gi * 16
            sv = src_b[pl.ds(o, 16)]
            dv = dst_b[pl.ds(o, 16)]
            tv = et_b[pl.ds(o, 16)]
            si0 = sv * 2
            di0 = dv * 2
            ti0 = tv * 2
            l0 = (plsc.load_gather(ssrc_v, [si0])
                  + plsc.load_gather(sdst_v, [di0])
                  + plsc.load_gather(sg_v, [ti0]))
            l1 = (plsc.load_gather(ssrc_v, [si0 + 1])
                  + plsc.load_gather(sdst_v, [di0 + 1])
                  + plsc.load_gather(sg_v, [ti0 + 1]))
            e0 = jnp.exp(_leaky(l0))
            e1 = jnp.exp(_leaky(l1))
            ev0_b[pl.ds(o, 16)] = e0
            ev1_b[pl.ds(o, 16)] = e1
            plsc.addupdate_scatter(den_v, [si0], e0)
            plsc.addupdate_scatter(den_v, [si0 + 1], e1)

        pltpu.sync_copy(ev0_b, ev0_hbm.at[pl.ds(b, CH1)])
        pltpu.sync_copy(ev1_b, ev1_hbm.at[pl.ds(b, CH1)])

    pltpu.sync_copy(den_v, red_sh.at[s])
    plsc.subcore_barrier()
    roff = s * RED
    pltpu.sync_copy(red_sh.at[0, pl.ds(roff, RED)], acc_v)

    @pl.loop(1, NS)
    def _red(j):
        pltpu.sync_copy(red_sh.at[j, pl.ds(roff, RED)], tmp_v)

        @pl.loop(0, RED // 16)
        def _acc(i):
            o = i * 16
            acc_v[pl.ds(o, 16)] = acc_v[pl.ds(o, 16)] + tmp_v[pl.ds(o, 16)]

    pltpu.sync_copy(acc_v, den_hbm.at[c, pl.ds(roff, RED)])


@functools.partial(
    pl.kernel,
    out_type=(
        jax.ShapeDtypeStruct((E,), _f32),
        jax.ShapeDtypeStruct((E,), _f32),
        jax.ShapeDtypeStruct((NC, DEN), _f32),
    ),
    mesh=plsc.VectorSubcoreMesh(core_axis_name="c", subcore_axis_name="s",
                                num_cores=NC, num_subcores=NS),
    compiler_params=pltpu.CompilerParams(needs_layout_passes=False),
    scratch_types=(
        pltpu.VMEM((DEN,), _f32),
        pltpu.VMEM((DEN,), _f32),
        pltpu.VMEM((CH1,), _i32),
        pltpu.VMEM((CH1,), _i32),
        pltpu.VMEM((CH1,), _f32),
        pltpu.VMEM((CH1,), _f32),
        pltpu.VMEM_SHARED((NS, DEN), _f32),
        pltpu.VMEM((RED,), _f32),
        pltpu.VMEM((RED,), _f32),
    ),
)
def _p1b(src_hbm, dst_hbm, ev0_hbm, ev1_hbm, inv_hbm,
         al0_hbm, al1_hbm, asum_hbm,
         inv_v, a_v, src_b, dst_b, e0_b, e1_b, red_sh, acc_v, tmp_v):
    c = lax.axis_index("c")
    s = lax.axis_index("s")
    wid = s * NC + c

    pltpu.sync_copy(inv_hbm, inv_v)
    zero16 = jnp.zeros((16,), _f32)

    @pl.loop(0, DEN // 16)
    def _zero(i):
        a_v[pl.ds(i * 16, 16)] = zero16

    base0 = wid * EPT

    @pl.loop(0, NCH1)
    def _chunk(ci):
        b = base0 + ci * CH1
        pltpu.sync_copy(src_hbm.at[pl.ds(b, CH1)], src_b)
        pltpu.sync_copy(dst_hbm.at[pl.ds(b, CH1)], dst_b)
        pltpu.sync_copy(ev0_hbm.at[pl.ds(b, CH1)], e0_b)
        pltpu.sync_copy(ev1_hbm.at[pl.ds(b, CH1)], e1_b)

        @pl.loop(0, NG1)
        def _grp(gi):
            o = gi * 16
            sv = src_b[pl.ds(o, 16)]
            dv = dst_b[pl.ds(o, 16)]
            si0 = sv * 2
            di0 = dv * 2
            a0 = e0_b[pl.ds(o, 16)] * plsc.load_gather(inv_v, [si0])
            a1 = e1_b[pl.ds(o, 16)] * plsc.load_gather(inv_v, [si0 + 1])
            e0_b[pl.ds(o, 16)] = a0
            e1_b[pl.ds(o, 16)] = a1
            plsc.addupdate_scatter(a_v, [di0], a0)
            plsc.addupdate_scatter(a_v, [di0 + 1], a1)

        pltpu.sync_copy(e0_b, al0_hbm.at[pl.ds(b, CH1)])
        pltpu.sync_copy(e1_b, al1_hbm.at[pl.ds(b, CH1)])

    pltpu.sync_copy(a_v, red_sh.at[s])
    plsc.subcore_barrier()
    roff = s * RED
    pltpu.sync_copy(red_sh.at[0, pl.ds(roff, RED)], acc_v)

    @pl.loop(1, NS)
    def _red(j):
        pltpu.sync_copy(red_sh.at[j, pl.ds(roff, RED)], tmp_v)

        @pl.loop(0, RED // 16)
        def _acc(i):
            o = i * 16
            acc_v[pl.ds(o, 16)] = acc_v[pl.ds(o, 16)] + tmp_v[pl.ds(o, 16)]

    pltpu.sync_copy(acc_v, asum_hbm.at[c, pl.ds(roff, RED)])


@functools.partial(
    pl.kernel,
    out_type=jax.ShapeDtypeStruct((NC, NPAD, D), _f32),
    mesh=plsc.VectorSubcoreMesh(core_axis_name="c", subcore_axis_name="s",
                                num_cores=NC, num_subcores=NS),
    compiler_params=pltpu.CompilerParams(needs_layout_passes=False),
    scratch_types=(
        pltpu.VMEM((CH2,), _i32),
        pltpu.VMEM((CH2,), _i32),
        pltpu.VMEM((CH2,), _i32),
        pltpu.VMEM((CH2,), _f32),
        pltpu.VMEM((CH2,), _f32),
        pltpu.VMEM((2 * CH2,), _i32),
        pltpu.VMEM((2 * CH2, D), _f32),
        pltpu.VMEM((CH2, D), _f32),
        pltpu.VMEM_SHARED((NPAD, D), _f32),
    ),
)
def _p2(src_hbm, dst_hbm, et_hbm, al0_hbm, al1_hbm, tab_hbm,
        outp_hbm,
        src_b, dst_b, et_b, al0_b, al1_b, idx_b, rows_v, contrib_v,
        acc_sh):
    c = lax.axis_index("c")
    s = lax.axis_index("s")
    wid = s * NC + c
    zero16 = jnp.zeros((16,), _f32)
    npad16 = jnp.full((16,), NPAD, _i32)

    @pl.loop(0, CH2)
    def _zc(i):
        for j in range(D // 16):
            contrib_v[i, pl.ds(j * 16, 16)] = zero16

    rbase = s * ROWS_PT

    @pl.loop(0, ROWS_PT // CH2)
    def _zacc(rr):
        pltpu.sync_copy(contrib_v, acc_sh.at[pl.ds(rbase + rr * CH2, CH2), :])

    plsc.subcore_barrier()

    nchunks = jnp.where(wid < NCH2 - (NCH2 // NW) * NW, NCH2 // NW + 1,
                        NCH2 // NW)

    @pl.loop(0, nchunks)
    def _chunk(k):
        b = (wid + k * NW) * CH2
        pltpu.sync_copy(src_hbm.at[pl.ds(b, CH2)], src_b)
        pltpu.sync_copy(dst_hbm.at[pl.ds(b, CH2)], dst_b)
        pltpu.sync_copy(et_hbm.at[pl.ds(b, CH2)], et_b)
        pltpu.sync_copy(al0_hbm.at[pl.ds(b, CH2)], al0_b)
        pltpu.sync_copy(al1_hbm.at[pl.ds(b, CH2)], al1_b)

        @pl.loop(0, CH2 // 16)
        def _bi(gi):
            o = gi * 16
            idx_b[pl.ds(o, 16)] = src_b[pl.ds(o, 16)]
            idx_b[pl.ds(CH2 + o, 16)] = et_b[pl.ds(o, 16)] + npad16

        pltpu.sync_copy(tab_hbm.at[idx_b], rows_v)

        @pl.loop(0, CH2)
        def _edge(e):
            eidx = jnp.full((16,), e, _i32)
            a0v = plsc.load_gather(al0_b, [eidx])
            a1v = plsc.load_gather(al1_b, [eidx])
            for j in range(D // 16):
                av = a0v if j < (D // 32) else a1v
                prj = rows_v[e, pl.ds(j * 16, 16)]
                pgj = rows_v[CH2 + e, pl.ds(j * 16, 16)]
                contrib_v[e, pl.ds(j * 16, 16)] = (prj + pgj) * av

        pltpu.sync_copy(contrib_v, acc_sh.at[dst_b], add=True)

    plsc.subcore_barrier()
    pltpu.sync_copy(acc_sh.at[pl.ds(rbase, ROWS_PT), :],
                    outp_hbm.at[c, pl.ds(rbase, ROWS_PT), :])


NB = 1000


def _k0_body(g_ref, wg_ref, bcat_ref, wt_ref, wrel_ref, brel_ref,
             pg4_ref, sg4_ref, gp_ref):
    g = g_ref[...]
    pg4 = jnp.dot(g, wg_ref[...], preferred_element_type=_f32,
                  precision=lax.Precision.HIGHEST) + bcat_ref[...][None, :]
    pg4_ref[...] = pg4
    sg4_ref[...] = jnp.dot(pg4, wt_ref[...], preferred_element_type=_f32,
                           precision=lax.Precision.HIGHEST)
    gp_ref[...] = jnp.dot(g, wrel_ref[...], preferred_element_type=_f32,
                          precision=lax.Precision.HIGHEST) + brel_ref[...][None, :]


def _tc_g_tables(g, wg_cat, b_cat, wt_big, wrel_t, b_rel):
    return pl.pallas_call(
        _k0_body,
        out_shape=(
            jax.ShapeDtypeStruct((R, 4 * D), _f32),
            jax.ShapeDtypeStruct((R, 8), _f32),
            jax.ShapeDtypeStruct((R, D), _f32),
        ),
    )(g, wg_cat, b_cat, wt_big, wrel_t, b_rel)


def _invden_body(di_ref, do_ref, ii_ref, io_ref):
    ii_ref[...] = 1.0 / (di_ref[0] + di_ref[1])
    io_ref[...] = 1.0 / (do_ref[0] + do_ref[1])


def _tc_invden(den_i, den_o):
    return pl.pallas_call(
        _invden_body,
        out_shape=(jax.ShapeDtypeStruct((DEN,), _f32),
                   jax.ShapeDtypeStruct((DEN,), _f32)),
    )(den_i, den_o)


def _norm_rows(v):
    nrm = jnp.sqrt(jnp.sum(v * v, axis=1, keepdims=True))
    return v / jnp.maximum(nrm, 1e-12)


def _k1_body(x_ref, w_ref, pri_ref, pci_ref, pro_ref, pco_ref, s8_ref,
             ent_ref):
    xn = _norm_rows(x_ref[...])
    big = jnp.dot(xn, w_ref[...], preferred_element_type=_f32,
                  precision=lax.Precision.HIGHEST)
    pri_ref[...] = big[:, 0:128]
    pci_ref[...] = big[:, 128:256]
    pro_ref[...] = big[:, 256:384]
    pco_ref[...] = big[:, 384:512]
    s8_ref[...] = big[:, 512:520]
    ent_ref[...] = big[:, 520:584]


def _tc_layer1(x, wcat1):
    blk = lambda w: pl.BlockSpec((NB, w), lambda i: (i, 0))
    return pl.pallas_call(
        _k1_body,
        grid=(N // NB,),
        in_specs=[blk(XS), pl.BlockSpec((XS, 584), lambda i: (0, 0))],
        out_specs=[blk(D), blk(D), blk(D), blk(D), blk(8), blk(HID)],
        out_shape=(
            jax.ShapeDtypeStruct((N, D), _f32),
            jax.ShapeDtypeStruct((N, D), _f32),
            jax.ShapeDtypeStruct((N, D), _f32),
            jax.ShapeDtypeStruct((N, D), _f32),
            jax.ShapeDtypeStruct((N, 8), _f32),
            jax.ShapeDtypeStruct((N, HID), _f32),
        ),
    )(x, wcat1)


def _combine(oi_ref, ai_ref, pci_ref, oo_ref, ao_ref, pco_ref):
    def one_dir(o_ref, a_ref, pc_ref):
        hsum = o_ref[0] + o_ref[1]
        a2 = a_ref[0] + a_ref[1]
        aexp = jnp.concatenate(
            [jnp.broadcast_to(a2[:, 0:1], (NB, HID)),
             jnp.broadcast_to(a2[:, 1:2], (NB, HID))], axis=1)
        return hsum + aexp * pc_ref[...]

    h = ALPHA * one_dir(oi_ref, ai_ref, pci_ref) + \
        (1.0 - ALPHA) * one_dir(oo_ref, ao_ref, pco_ref)
    h = _leaky(h)
    h0 = _norm_rows(h[:, 0:HID])
    h1 = _norm_rows(h[:, HID:D])
    return jnp.concatenate([h0, h1], axis=1)


def _k2_body(oi_ref, ai_ref, pci_ref, oo_ref, ao_ref, pco_ref, w_ref,
             pri_ref, pci2_ref, pro_ref, pco2_ref, s8_ref):
    h = _combine(oi_ref, ai_ref, pci_ref, oo_ref, ao_ref, pco_ref)
    big = jnp.dot(h, w_ref[...], preferred_element_type=_f32,
                  precision=lax.Precision.HIGHEST)
    pri_ref[...] = big[:, 0:128]
    pci2_ref[...] = big[:, 128:256]
    pro_ref[...] = big[:, 256:384]
    pco2_ref[...] = big[:, 384:512]
    s8_ref[...] = big[:, 512:520]


def _tc_layer2(outp_i, a_i, pc_i, outp_o, a_o, pc_o, wcat2):
    blk = lambda w: pl.BlockSpec((NB, w), lambda i: (i, 0))
    blk3 = pl.BlockSpec((NC, NB, D), lambda i: (0, i, 0))
    blka = pl.BlockSpec((NC, NB, 2), lambda i: (0, i, 0))
    return pl.pallas_call(
        _k2_body,
        grid=(N // NB,),
        in_specs=[blk3, blka, blk(D), blk3, blka, blk(D),
                  pl.BlockSpec((D, 520), lambda i: (0, 0))],
        out_specs=[blk(D), blk(D), blk(D), blk(D), blk(8)],
        out_shape=(
            jax.ShapeDtypeStruct((N, D), _f32),
            jax.ShapeDtypeStruct((N, D), _f32),
            jax.ShapeDtypeStruct((N, D), _f32),
            jax.ShapeDtypeStruct((N, D), _f32),
            jax.ShapeDtypeStruct((N, 8), _f32),
        ),
    )(outp_i, a_i, pc_i, outp_o, a_o, pc_o, wcat2)


def _k3_body(oi_ref, ai_ref, pci_ref, oo_ref, ao_ref, pco_ref, ent_ref,
             bent_ref, out_ref):
    h = _combine(oi_ref, ai_ref, pci_ref, oo_ref, ao_ref, pco_ref)
    ent = ent_ref[...] + bent_ref[...][None, :]
    hp = h + jnp.concatenate([ent, ent], axis=1)
    out_ref[...] = _norm_rows(hp)


def _tc_final(outp_i, a_i, pc_i, outp_o, a_o, pc_o, ent, b_ent):
    blk = lambda w: pl.BlockSpec((NB, w), lambda i: (i, 0))
    blk3 = pl.BlockSpec((NC, NB, D), lambda i: (0, i, 0))
    blka = pl.BlockSpec((NC, NB, 2), lambda i: (0, i, 0))
    return pl.pallas_call(
        _k3_body,
        grid=(N // NB,),
        in_specs=[blk3, blka, blk(D), blk3, blka, blk(D), blk(HID),
                  pl.BlockSpec((HID,), lambda i: (0,))],
        out_specs=blk(D),
        out_shape=jax.ShapeDtypeStruct((N, D), _f32),
    )(outp_i, a_i, pc_i, outp_o, a_o, pc_o, ent, b_ent)


def _wt_block(att):
    z = jnp.zeros((HID, 1), _f32)
    c0 = jnp.concatenate([att[0, 0][:, None], z], axis=0)
    c1 = jnp.concatenate([z, att[0, 1][:, None]], axis=0)
    return jnp.concatenate([c0, c1], axis=1)


def _split_w(W):
    dh = (W.shape[1] - GS) // 2
    return W[:, 0:dh].T, W[:, dh:2 * dh].T, W[:, 2 * dh:].T


def _layer_tables(wt_in, wt_out, W_in, W_out):
    wr_i, wc_i, _ = _split_w(W_in)
    wr_o, wc_o, _ = _split_w(W_out)
    cols = [wr_i, wc_i, wr_o, wc_o,
            wr_i @ wt_in, wc_i @ wt_in, wr_o @ wt_out, wc_o @ wt_out]
    return jnp.concatenate(cols, axis=1)


def _mk_tab(pr, pg):
    top = jnp.concatenate([pr, jnp.zeros((NPAD - N, D), _f32)], axis=0)
    bot = jnp.concatenate(
        [pg, jnp.zeros((TABR - NPAD - R, D), _f32)], axis=0)
    return jnp.concatenate([top, bot], axis=0)


def _run_sc_layer(row, col, et, s8, sg4, sg_cols, pr_i, pc_i, pr_o, pc_o,
                  pg_i, pg_o):
    def _padto(v, L):
        return jnp.concatenate([v, jnp.zeros((L - v.shape[0],), _f32)])

    sr_i = _padto(s8[:, 0:2].reshape(-1), DEN)
    sc_i = _padto(s8[:, 2:4].reshape(-1), DEN)
    sr_o = _padto(s8[:, 4:6].reshape(-1), DEN)
    sc_o = _padto(s8[:, 6:8].reshape(-1), DEN)
    sg_i = _padto(sg4[:, sg_cols[0]:sg_cols[0] + 2].reshape(-1), 512)
    sg_o = _padto(sg4[:, sg_cols[1]:sg_cols[1] + 2].reshape(-1), 512)

    ev0_i, ev1_i, den_i = _p1(row, col, et, sr_i, sc_i, sg_i)
    ev0_o, ev1_o, den_o = _p1(col, row, et, sr_o, sc_o, sg_o)
    inv_i, inv_o = _tc_invden(den_i, den_o)
    al0_i, al1_i, a_i = _p1b(row, col, ev0_i, ev1_i, inv_i)
    al0_o, al1_o, a_o = _p1b(col, row, ev0_o, ev1_o, inv_o)
    outp_i = _p2(row, col, et, al0_i, al1_i, _mk_tab(pr_i, pg_i))
    outp_o = _p2(col, row, et, al0_o, al1_o, _mk_tab(pr_o, pg_o))
    return (outp_i, a_i.reshape(NC, NPAD, 2), pc_i,
            outp_o, a_o.reshape(NC, NPAD, 2), pc_o)


def kernel(x, g, edge_idx, edge_type, W_in1, b_in1, att_in1, W_out1, b_out1,
           att_out1, W_in2, b_in2, att_in2, W_out2, b_out2, att_out2,
           W_ent, b_ent, W_rel, b_rel):
    row = edge_idx[0]
    col = edge_idx[1]
    et = edge_type

    wt_in1 = _wt_block(att_in1)
    wt_out1 = _wt_block(att_out1)
    wt_in2 = _wt_block(att_in2)
    wt_out2 = _wt_block(att_out2)

    wcat1 = jnp.concatenate(
        [_layer_tables(wt_in1, wt_out1, W_in1, W_out1), W_ent.T], axis=1)
    wcat2 = _layer_tables(wt_in2, wt_out2, W_in2, W_out2)

    _, _, wg_i1 = _split_w(W_in1)
    _, _, wg_o1 = _split_w(W_out1)
    _, _, wg_i2 = _split_w(W_in2)
    _, _, wg_o2 = _split_w(W_out2)
    wg_cat = jnp.concatenate([wg_i1, wg_o1, wg_i2, wg_o2], axis=1)
    b_cat = jnp.concatenate([b_in1, b_out1, b_in2, b_out2])
    z2 = jnp.zeros((D, 2), _f32)

    def blkdiag(w0, w1, w2, w3):
        def rowblk(i, w):
            pre = [z2] * i
            post = [z2] * (3 - i)
            return jnp.concatenate(pre + [w] + post, axis=1)
        return jnp.concatenate(
            [rowblk(0, w0), rowblk(1, w1), rowblk(2, w2), rowblk(3, w3)],
            axis=0)

    wt_big = blkdiag(wt_in1, wt_out1, wt_in2, wt_out2)

    pg4, sg4, g_prime = _tc_g_tables(g, wg_cat, b_cat, wt_big, W_rel.T, b_rel)

    pr_i1, pc_i1, pr_o1, pc_o1, s8_1, ent = _tc_layer1(x, wcat1)
    sc_args1 = _run_sc_layer(
        row, col, et, s8_1, sg4, (0, 2), pr_i1, pc_i1, pr_o1, pc_o1,
        pg4[:, 0:128], pg4[:, 128:256])

    pr_i2, pc_i2, pr_o2, pc_o2, s8_2 = _tc_layer2(*sc_args1, wcat2)
    sc_args2 = _run_sc_layer(
        row, col, et, s8_2, sg4, (4, 6), pr_i2, pc_i2, pr_o2, pc_o2,
        pg4[:, 256:384], pg4[:, 384:512])

    h_prime = _tc_final(*sc_args2, ent, b_ent)
    return (h_prime, g_prime)


# speedup vs baseline: 1.1575x; 1.0603x over previous
import functools

import jax
import jax.numpy as jnp
from jax import lax
from jax.experimental import pallas as pl
from jax.experimental.pallas import tpu as pltpu
from jax.experimental.pallas import tpu_sc as plsc

N = 10000
E = 320000
XS = 128
GS = 16
R = 200
HEADS = 2
HID = 64
OUT = 64
ALPHA = 0.5
D = HEADS * HID

NC = 2
NS = 16
NW = NC * NS

NPAD = 10240
DEN = 2 * NPAD
RED = DEN // NS

EPT = E // NW
CH1 = 2000
NCH1 = EPT // CH1
NG1 = CH1 // 16

CH2 = 80
NCH2 = E // CH2
ROWS_PT = NPAD // NS
TABR = NPAD + 256

_f32 = jnp.float32
_i32 = jnp.int32


def _leaky(v):
    return jnp.where(v >= 0, v, 0.01 * v)


@functools.partial(
    pl.kernel,
    out_type=(
        jax.ShapeDtypeStruct((E,), _f32),
        jax.ShapeDtypeStruct((E,), _f32),
        jax.ShapeDtypeStruct((NC, DEN), _f32),
    ),
    mesh=plsc.VectorSubcoreMesh(core_axis_name="c", subcore_axis_name="s",
                                num_cores=NC, num_subcores=NS),
    compiler_params=pltpu.CompilerParams(needs_layout_passes=False),
    scratch_types=(
        pltpu.VMEM((DEN,), _f32),
        pltpu.VMEM((DEN,), _f32),
        pltpu.VMEM((512,), _f32),
        pltpu.VMEM((DEN,), _f32),
        pltpu.VMEM((CH1,), _i32),
        pltpu.VMEM((CH1,), _i32),
        pltpu.VMEM((CH1,), _i32),
        pltpu.VMEM((CH1,), _f32),
        pltpu.VMEM((CH1,), _f32),
        pltpu.VMEM_SHARED((NS, DEN), _f32),
        pltpu.VMEM((RED,), _f32),
        pltpu.VMEM((RED,), _f32),
    ),
)
def _p1(src_hbm, dst_hbm, et_hbm, ssrc_hbm, sdst_hbm, sg_hbm,
        ev0_hbm, ev1_hbm, den_hbm,
        ssrc_v, sdst_v, sg_v, den_v, src_b, dst_b, et_b, ev0_b, ev1_b,
        red_sh, acc_v, tmp_v):
    c = lax.axis_index("c")
    s = lax.axis_index("s")
    wid = s * NC + c

    pltpu.sync_copy(ssrc_hbm, ssrc_v)
    pltpu.sync_copy(sdst_hbm, sdst_v)
    pltpu.sync_copy(sg_hbm, sg_v)

    zero16 = jnp.zeros((16,), _f32)

    @pl.loop(0, DEN // 16)
    def _zero(i):
        den_v[pl.ds(i * 16, 16)] = zero16

    base0 = wid * EPT

    @pl.loop(0, NCH1)
    def _chunk(ci):
        b = base0 + ci * CH1
        pltpu.sync_copy(src_hbm.at[pl.ds(b, CH1)], src_b)
        pltpu.sync_copy(dst_hbm.at[pl.ds(b, CH1)], dst_b)
        pltpu.sync_copy(et_hbm.at[pl.ds(b, CH1)], et_b)

        @pl.loop(0, NG1)
        def _grp(gi):
            o = gi * 16
            sv = src_b[pl.ds(o, 16)]
            dv = dst_b[pl.ds(o, 16)]
            tv = et_b[pl.ds(o, 16)]
            si0 = sv * 2
            di0 = dv * 2
            ti0 = tv * 2
            l0 = (plsc.load_gather(ssrc_v, [si0])
                  + plsc.load_gather(sdst_v, [di0])
                  + plsc.load_gather(sg_v, [ti0]))
            l1 = (plsc.load_gather(ssrc_v, [si0 + 1])
                  + plsc.load_gather(sdst_v, [di0 + 1])
                  + plsc.load_gather(sg_v, [ti0 + 1]))
            e0 = jnp.exp(_leaky(l0))
            e1 = jnp.exp(_leaky(l1))
            ev0_b[pl.ds(o, 16)] = e0
            ev1_b[pl.ds(o, 16)] = e1
            plsc.addupdate_scatter(den_v, [si0], e0)
            plsc.addupdate_scatter(den_v, [si0 + 1], e1)

        pltpu.sync_copy(ev0_b, ev0_hbm.at[pl.ds(b, CH1)])
        pltpu.sync_copy(ev1_b, ev1_hbm.at[pl.ds(b, CH1)])

    pltpu.sync_copy(den_v, red_sh.at[s])
    plsc.subcore_barrier()
    roff = s * RED
    pltpu.sync_copy(red_sh.at[0, pl.ds(roff, RED)], acc_v)

    @pl.loop(1, NS)
    def _red(j):
        pltpu.sync_copy(red_sh.at[j, pl.ds(roff, RED)], tmp_v)

        @pl.loop(0, RED // 16)
        def _acc(i):
            o = i * 16
            acc_v[pl.ds(o, 16)] = acc_v[pl.ds(o, 16)] + tmp_v[pl.ds(o, 16)]

    pltpu.sync_copy(acc_v, den_hbm.at[c, pl.ds(roff, RED)])


@functools.partial(
    pl.kernel,
    out_type=(
        jax.ShapeDtypeStruct((E,), _f32),
        jax.ShapeDtypeStruct((E,), _f32),
        jax.ShapeDtypeStruct((NC, DEN), _f32),
    ),
    mesh=plsc.VectorSubcoreMesh(core_axis_name="c", subcore_axis_name="s",
                                num_cores=NC, num_subcores=NS),
    compiler_params=pltpu.CompilerParams(needs_layout_passes=False),
    scratch_types=(
        pltpu.VMEM((DEN,), _f32),
        pltpu.VMEM((DEN,), _f32),
        pltpu.VMEM((CH1,), _i32),
        pltpu.VMEM((CH1,), _i32),
        pltpu.VMEM((CH1,), _f32),
        pltpu.VMEM((CH1,), _f32),
        pltpu.VMEM_SHARED((NS, DEN), _f32),
        pltpu.VMEM((RED,), _f32),
        pltpu.VMEM((RED,), _f32),
    ),
)
def _p1b(src_hbm, dst_hbm, ev0_hbm, ev1_hbm, inv_hbm,
         al0_hbm, al1_hbm, asum_hbm,
         inv_v, a_v, src_b, dst_b, e0_b, e1_b, red_sh, acc_v, tmp_v):
    c = lax.axis_index("c")
    s = lax.axis_index("s")
    wid = s * NC + c

    pltpu.sync_copy(inv_hbm, inv_v)
    zero16 = jnp.zeros((16,), _f32)

    @pl.loop(0, DEN // 16)
    def _zero(i):
        a_v[pl.ds(i * 16, 16)] = zero16

    base0 = wid * EPT

    @pl.loop(0, NCH1)
    def _chunk(ci):
        b = base0 + ci * CH1
        pltpu.sync_copy(src_hbm.at[pl.ds(b, CH1)], src_b)
        pltpu.sync_copy(dst_hbm.at[pl.ds(b, CH1)], dst_b)
        pltpu.sync_copy(ev0_hbm.at[pl.ds(b, CH1)], e0_b)
        pltpu.sync_copy(ev1_hbm.at[pl.ds(b, CH1)], e1_b)

        @pl.loop(0, NG1)
        def _grp(gi):
            o = gi * 16
            sv = src_b[pl.ds(o, 16)]
            dv = dst_b[pl.ds(o, 16)]
            si0 = sv * 2
            di0 = dv * 2
            a0 = e0_b[pl.ds(o, 16)] * plsc.load_gather(inv_v, [si0])
            a1 = e1_b[pl.ds(o, 16)] * plsc.load_gather(inv_v, [si0 + 1])
            e0_b[pl.ds(o, 16)] = a0
            e1_b[pl.ds(o, 16)] = a1
            plsc.addupdate_scatter(a_v, [di0], a0)
            plsc.addupdate_scatter(a_v, [di0 + 1], a1)

        pltpu.sync_copy(e0_b, al0_hbm.at[pl.ds(b, CH1)])
        pltpu.sync_copy(e1_b, al1_hbm.at[pl.ds(b, CH1)])

    pltpu.sync_copy(a_v, red_sh.at[s])
    plsc.subcore_barrier()
    roff = s * RED
    pltpu.sync_copy(red_sh.at[0, pl.ds(roff, RED)], acc_v)

    @pl.loop(1, NS)
    def _red(j):
        pltpu.sync_copy(red_sh.at[j, pl.ds(roff, RED)], tmp_v)

        @pl.loop(0, RED // 16)
        def _acc(i):
            o = i * 16
            acc_v[pl.ds(o, 16)] = acc_v[pl.ds(o, 16)] + tmp_v[pl.ds(o, 16)]

    pltpu.sync_copy(acc_v, asum_hbm.at[c, pl.ds(roff, RED)])


@functools.partial(
    pl.kernel,
    out_type=jax.ShapeDtypeStruct((NC, NPAD, D), _f32),
    mesh=plsc.VectorSubcoreMesh(core_axis_name="c", subcore_axis_name="s",
                                num_cores=NC, num_subcores=NS),
    compiler_params=pltpu.CompilerParams(needs_layout_passes=False),
    scratch_types=(
        pltpu.VMEM((CH2,), _i32),
        pltpu.VMEM((CH2,), _i32),
        pltpu.VMEM((CH2,), _i32),
        pltpu.VMEM((CH2,), _f32),
        pltpu.VMEM((CH2,), _f32),
        pltpu.VMEM((2 * CH2,), _i32),
        pltpu.VMEM((2 * CH2, D), _f32),
        pltpu.VMEM((CH2, D), _f32),
        pltpu.VMEM_SHARED((NPAD, D), _f32),
    ),
)
def _p2(src_hbm, dst_hbm, et_hbm, al0_hbm, al1_hbm, tab_hbm,
        outp_hbm,
        src_b, dst_b, et_b, al0_b, al1_b, idx_b, rows_v, contrib_v,
        acc_sh):
    c = lax.axis_index("c")
    s = lax.axis_index("s")
    wid = s * NC + c
    zero16 = jnp.zeros((16,), _f32)
    npad16 = jnp.full((16,), NPAD, _i32)

    @pl.loop(0, CH2)
    def _zc(i):
        for j in range(D // 16):
            contrib_v[i, pl.ds(j * 16, 16)] = zero16

    rbase = s * ROWS_PT

    @pl.loop(0, ROWS_PT // CH2)
    def _zacc(rr):
        pltpu.sync_copy(contrib_v, acc_sh.at[pl.ds(rbase + rr * CH2, CH2), :])

    plsc.subcore_barrier()

    nchunks = jnp.where(wid < NCH2 - (NCH2 // NW) * NW, NCH2 // NW + 1,
                        NCH2 // NW)

    @pl.loop(0, nchunks)
    def _chunk(k):
        b = (wid + k * NW) * CH2
        pltpu.sync_copy(src_hbm.at[pl.ds(b, CH2)], src_b)
        pltpu.sync_copy(dst_hbm.at[pl.ds(b, CH2)], dst_b)
        pltpu.sync_copy(et_hbm.at[pl.ds(b, CH2)], et_b)
        pltpu.sync_copy(al0_hbm.at[pl.ds(b, CH2)], al0_b)
        pltpu.sync_copy(al1_hbm.at[pl.ds(b, CH2)], al1_b)

        @pl.loop(0, CH2 // 16)
        def _bi(gi):
            o = gi * 16
            idx_b[pl.ds(o, 16)] = src_b[pl.ds(o, 16)]
            idx_b[pl.ds(CH2 + o, 16)] = et_b[pl.ds(o, 16)] + npad16

        pltpu.sync_copy(tab_hbm.at[idx_b], rows_v)

        @pl.loop(0, CH2)
        def _edge(e):
            eidx = jnp.full((16,), e, _i32)
            a0v = plsc.load_gather(al0_b, [eidx])
            a1v = plsc.load_gather(al1_b, [eidx])
            for j in range(D // 16):
                av = a0v if j < (D // 32) else a1v
                prj = rows_v[e, pl.ds(j * 16, 16)]
                pgj = rows_v[CH2 + e, pl.ds(j * 16, 16)]
                contrib_v[e, pl.ds(j * 16, 16)] = (prj + pgj) * av

        pltpu.sync_copy(contrib_v, acc_sh.at[dst_b], add=True)

    plsc.subcore_barrier()
    pltpu.sync_copy(acc_sh.at[pl.ds(rbase, ROWS_PT), :],
                    outp_hbm.at[c, pl.ds(rbase, ROWS_PT), :])


NB = 1000


def _k0_body(g_ref, wg_ref, bcat_ref, wt_ref, wrel_ref, brel_ref,
             pg4_ref, sg4_ref, gp_ref):
    g = g_ref[...]
    pg4 = jnp.dot(g, wg_ref[...], preferred_element_type=_f32,
                  precision=lax.Precision.HIGHEST) + bcat_ref[...][None, :]
    pg4_ref[...] = pg4
    sg4_ref[...] = jnp.dot(pg4, wt_ref[...], preferred_element_type=_f32,
                           precision=lax.Precision.HIGHEST)
    gp_ref[...] = jnp.dot(g, wrel_ref[...], preferred_element_type=_f32,
                          precision=lax.Precision.HIGHEST) + brel_ref[...][None, :]


def _tc_g_tables(g, wg_cat, b_cat, wt_big, wrel_t, b_rel):
    return pl.pallas_call(
        _k0_body,
        out_shape=(
            jax.ShapeDtypeStruct((R, 4 * D), _f32),
            jax.ShapeDtypeStruct((R, 8), _f32),
            jax.ShapeDtypeStruct((R, D), _f32),
        ),
    )(g, wg_cat, b_cat, wt_big, wrel_t, b_rel)


def _invden_body(di_ref, do_ref, ii_ref, io_ref):
    ii_ref[...] = 1.0 / (di_ref[0] + di_ref[1])
    io_ref[...] = 1.0 / (do_ref[0] + do_ref[1])


def _tc_invden(den_i, den_o):
    return pl.pallas_call(
        _invden_body,
        out_shape=(jax.ShapeDtypeStruct((DEN,), _f32),
                   jax.ShapeDtypeStruct((DEN,), _f32)),
    )(den_i, den_o)


def _norm_rows(v):
    nrm = jnp.sqrt(jnp.sum(v * v, axis=1, keepdims=True))
    return v / jnp.maximum(nrm, 1e-12)


def _k1_body(x_ref, w_ref, pri_ref, pci_ref, pro_ref, pco_ref, s8_ref,
             ent_ref):
    xn = _norm_rows(x_ref[...])
    big = jnp.dot(xn, w_ref[...], preferred_element_type=_f32,
                  precision=lax.Precision.HIGHEST)
    pri_ref[...] = big[:, 0:128]
    pci_ref[...] = big[:, 128:256]
    pro_ref[...] = big[:, 256:384]
    pco_ref[...] = big[:, 384:512]
    s8_ref[...] = big[:, 512:520]
    ent_ref[...] = big[:, 520:584]


def _tc_layer1(x, wcat1):
    blk = lambda w: pl.BlockSpec((NB, w), lambda i: (i, 0))
    return pl.pallas_call(
        _k1_body,
        grid=(N // NB,),
        in_specs=[blk(XS), pl.BlockSpec((XS, 584), lambda i: (0, 0))],
        out_specs=[blk(D), blk(D), blk(D), blk(D), blk(8), blk(HID)],
        out_shape=(
            jax.ShapeDtypeStruct((N, D), _f32),
            jax.ShapeDtypeStruct((N, D), _f32),
            jax.ShapeDtypeStruct((N, D), _f32),
            jax.ShapeDtypeStruct((N, D), _f32),
            jax.ShapeDtypeStruct((N, 8), _f32),
            jax.ShapeDtypeStruct((N, HID), _f32),
        ),
    )(x, wcat1)


def _combine(oi_ref, ai_ref, pci_ref, oo_ref, ao_ref, pco_ref):
    def one_dir(o_ref, a_ref, pc_ref):
        hsum = o_ref[0] + o_ref[1]
        a2 = a_ref[0] + a_ref[1]
        aexp = jnp.concatenate(
            [jnp.broadcast_to(a2[:, 0:1], (NB, HID)),
             jnp.broadcast_to(a2[:, 1:2], (NB, HID))], axis=1)
        return hsum + aexp * pc_ref[...]

    h = ALPHA * one_dir(oi_ref, ai_ref, pci_ref) + \
        (1.0 - ALPHA) * one_dir(oo_ref, ao_ref, pco_ref)
    h = _leaky(h)
    h0 = _norm_rows(h[:, 0:HID])
    h1 = _norm_rows(h[:, HID:D])
    return jnp.concatenate([h0, h1], axis=1)


def _k2_body(oi_ref, ai_ref, pci_ref, oo_ref, ao_ref, pco_ref, w_ref,
             pri_ref, pci2_ref, pro_ref, pco2_ref, s8_ref):
    h = _combine(oi_ref, ai_ref, pci_ref, oo_ref, ao_ref, pco_ref)
    big = jnp.dot(h, w_ref[...], preferred_element_type=_f32,
                  precision=lax.Precision.HIGHEST)
    pri_ref[...] = big[:, 0:128]
    pci2_ref[...] = big[:, 128:256]
    pro_ref[...] = big[:, 256:384]
    pco2_ref[...] = big[:, 384:512]
    s8_ref[...] = big[:, 512:520]


def _tc_layer2(outp_i, a_i, pc_i, outp_o, a_o, pc_o, wcat2):
    blk = lambda w: pl.BlockSpec((NB, w), lambda i: (i, 0))
    blk3 = pl.BlockSpec((NC, NB, D), lambda i: (0, i, 0))
    blka = pl.BlockSpec((NC, NB, 2), lambda i: (0, i, 0))
    return pl.pallas_call(
        _k2_body,
        grid=(N // NB,),
        in_specs=[blk3, blka, blk(D), blk3, blka, blk(D),
                  pl.BlockSpec((D, 520), lambda i: (0, 0))],
        out_specs=[blk(D), blk(D), blk(D), blk(D), blk(8)],
        out_shape=(
            jax.ShapeDtypeStruct((N, D), _f32),
            jax.ShapeDtypeStruct((N, D), _f32),
            jax.ShapeDtypeStruct((N, D), _f32),
            jax.ShapeDtypeStruct((N, D), _f32),
            jax.ShapeDtypeStruct((N, 8), _f32),
        ),
    )(outp_i, a_i, pc_i, outp_o, a_o, pc_o, wcat2)


def _k3_body(oi_ref, ai_ref, pci_ref, oo_ref, ao_ref, pco_ref, ent_ref,
             bent_ref, out_ref):
    h = _combine(oi_ref, ai_ref, pci_ref, oo_ref, ao_ref, pco_ref)
    ent = ent_ref[...] + bent_ref[...][None, :]
    hp = h + jnp.concatenate([ent, ent], axis=1)
    out_ref[...] = _norm_rows(hp)


def _tc_final(outp_i, a_i, pc_i, outp_o, a_o, pc_o, ent, b_ent):
    blk = lambda w: pl.BlockSpec((NB, w), lambda i: (i, 0))
    blk3 = pl.BlockSpec((NC, NB, D), lambda i: (0, i, 0))
    blka = pl.BlockSpec((NC, NB, 2), lambda i: (0, i, 0))
    return pl.pallas_call(
        _k3_body,
        grid=(N // NB,),
        in_specs=[blk3, blka, blk(D), blk3, blka, blk(D), blk(HID),
                  pl.BlockSpec((HID,), lambda i: (0,))],
        out_specs=blk(D),
        out_shape=jax.ShapeDtypeStruct((N, D), _f32),
    )(outp_i, a_i, pc_i, outp_o, a_o, pc_o, ent, b_ent)


def _wt_block(att):
    z = jnp.zeros((HID, 1), _f32)
    c0 = jnp.concatenate([att[0, 0][:, None], z], axis=0)
    c1 = jnp.concatenate([z, att[0, 1][:, None]], axis=0)
    return jnp.concatenate([c0, c1], axis=1)


def _split_w(W):
    dh = (W.shape[1] - GS) // 2
    return W[:, 0:dh].T, W[:, dh:2 * dh].T, W[:, 2 * dh:].T


def _layer_tables(wt_in, wt_out, W_in, W_out):
    wr_i, wc_i, _ = _split_w(W_in)
    wr_o, wc_o, _ = _split_w(W_out)
    cols = [wr_i, wc_i, wr_o, wc_o,
            wr_i @ wt_in, wc_i @ wt_in, wr_o @ wt_out, wc_o @ wt_out]
    return jnp.concatenate(cols, axis=1)


def _mk_tab(pr, pg):
    top = jnp.concatenate([pr, jnp.zeros((NPAD - N, D), _f32)], axis=0)
    bot = jnp.concatenate(
        [pg, jnp.zeros((TABR - NPAD - R, D), _f32)], axis=0)
    return jnp.concatenate([top, bot], axis=0)


def _run_sc_layer(row, col, et, s8, sg4, sg_cols, pr_i, pc_i, pr_o, pc_o,
                  pg_i, pg_o):
    def _padto(v, L):
        return jnp.concatenate([v, jnp.zeros((L - v.shape[0],), _f32)])

    sr_i = _padto(s8[:, 0:2].reshape(-1), DEN)
    sc_i = _padto(s8[:, 2:4].reshape(-1), DEN)
    sr_o = _padto(s8[:, 4:6].reshape(-1), DEN)
    sc_o = _padto(s8[:, 6:8].reshape(-1), DEN)
    sg_i = _padto(sg4[:, sg_cols[0]:sg_cols[0] + 2].reshape(-1), 512)
    sg_o = _padto(sg4[:, sg_cols[1]:sg_cols[1] + 2].reshape(-1), 512)

    ev0_i, ev1_i, den_i = _p1(row, col, et, sr_i, sc_i, sg_i)
    ev0_o, ev1_o, den_o = _p1(col, row, et, sr_o, sc_o, sg_o)
    inv_i, inv_o = _tc_invden(den_i, den_o)
    al0_i, al1_i, a_i = _p1b(row, col, ev0_i, ev1_i, inv_i)
    al0_o, al1_o, a_o = _p1b(col, row, ev0_o, ev1_o, inv_o)
    outp_i = _p2(row, col, et, al0_i, al1_i, _mk_tab(pr_i, pg_i))
    outp_o = _p2(col, row, et, al0_o, al1_o, _mk_tab(pr_o, pg_o))
    return (outp_i, a_i.reshape(NC, NPAD, 2), pc_i,
            outp_o, a_o.reshape(NC, NPAD, 2), pc_o)


def kernel(x, g, edge_idx, edge_type, W_in1, b_in1, att_in1, W_out1, b_out1,
           att_out1, W_in2, b_in2, att_in2, W_out2, b_out2, att_out2,
           W_ent, b_ent, W_rel, b_rel):
    row = edge_idx[0]
    col = edge_idx[1]
    et = edge_type

    wt_in1 = _wt_block(att_in1)
    wt_out1 = _wt_block(att_out1)
    wt_in2 = _wt_block(att_in2)
    wt_out2 = _wt_block(att_out2)

    wcat1 = jnp.concatenate(
        [_layer_tables(wt_in1, wt_out1, W_in1, W_out1), W_ent.T], axis=1)
    wcat2 = _layer_tables(wt_in2, wt_out2, W_in2, W_out2)

    _, _, wg_i1 = _split_w(W_in1)
    _, _, wg_o1 = _split_w(W_out1)
    _, _, wg_i2 = _split_w(W_in2)
    _, _, wg_o2 = _split_w(W_out2)
    wg_cat = jnp.concatenate([wg_i1, wg_o1, wg_i2, wg_o2], axis=1)
    b_cat = jnp.concatenate([b_in1, b_out1, b_in2, b_out2])
    z2 = jnp.zeros((D, 2), _f32)

    def blkdiag(w0, w1, w2, w3):
        def rowblk(i, w):
            pre = [z2] * i
            post = [z2] * (3 - i)
            return jnp.concatenate(pre + [w] + post, axis=1)
        return jnp.concatenate(
            [rowblk(0, w0), rowblk(1, w1), rowblk(2, w2), rowblk(3, w3)],
            axis=0)

    wt_big = blkdiag(wt_in1, wt_out1, wt_in2, wt_out2)

    pg4, sg4, g_prime = _tc_g_tables(g, wg_cat, b_cat, wt_big, W_rel.T, b_rel)

    pr_i1, pc_i1, pr_o1, pc_o1, s8_1, ent = _tc_layer1(x, wcat1)
    sc_args1 = _run_sc_layer(
        row, col, et, s8_1, sg4, (0, 2), pr_i1, pc_i1, pr_o1, pc_o1,
        pg4[:, 0:128], pg4[:, 128:256])

    pr_i2, pc_i2, pr_o2, pc_o2, s8_2 = _tc_layer2(*sc_args1, wcat2)
    sc_args2 = _run_sc_layer(
        row, col, et, s8_2, sg4, (4, 6), pr_i2, pc_i2, pr_o2, pc_o2,
        pg4[:, 256:384], pg4[:, 384:512])

    h_prime = _tc_final(*sc_args2, ent, b_ent)
    return (h_prime, g_prime)
